# baseline (speedup 1.0000x reference)
import functools

import jax
import jax.numpy as jnp
from jax import lax
from jax.experimental import pallas as pl
from jax.experimental.pallas import tpu as pltpu
from jax.experimental.pallas import tpu_sc as plsc

B = 4
S = 2048
D = 768
L = 16
NC = 2
NS = 16
NW = NC * NS
SPW = S // NW
HC = 32
NCH = B * SPW // HC
DV = D // L

_mesh = plsc.VectorSubcoreMesh(core_axis_name="c", subcore_axis_name="s")


@functools.partial(
    pl.kernel,
    mesh=_mesh,
    out_type=jax.ShapeDtypeStruct((B * S, D), jnp.float32),
    scratch_types=[
        pltpu.VMEM((B * SPW,), jnp.int32),
        pltpu.VMEM((B * SPW,), jnp.int32),
        pltpu.VMEM((2, HC, D), jnp.float32),
        pltpu.VMEM((SPW, D), jnp.float32),
        pltpu.VMEM((2, D), jnp.float32),
        pltpu.VMEM((D,), jnp.float32),
        pltpu.SemaphoreType.DMA,
        pltpu.SemaphoreType.DMA,
        pltpu.SemaphoreType.DMA,
        pltpu.SemaphoreType.DMA,
    ],
)
def _emb_kernel(ids_hbm, tt_hbm, word_hbm, pos_hbm, type_hbm, out_hbm,
                ids_v, tt_v, w_v, p2_v, tb_v, dd_v,
                g0, g1, o0, o1):
    gsem = (g0, g1)
    osem = (o0, o1)
    wid = lax.axis_index("s") * NC + lax.axis_index("c")
    s0 = wid * SPW

    for b in range(B):
        pltpu.sync_copy(ids_hbm.at[pl.ds(b * S + s0, SPW)],
                        ids_v.at[pl.ds(b * SPW, SPW)])
        pltpu.sync_copy(tt_hbm.at[pl.ds(b * S + s0, SPW)],
                        tt_v.at[pl.ds(b * SPW, SPW)])
    pltpu.sync_copy(pos_hbm.at[pl.ds(s0, SPW), :], p2_v)
    pltpu.sync_copy(type_hbm, tb_v)

    for d in range(DV):
        dsl = pl.ds(d * L, L)
        dd_v[dsl] = tb_v[1, dsl] - tb_v[0, dsl]

    def bias_body(k, _):
        for d in range(DV):
            dsl = pl.ds(d * L, L)
            p2_v[k, dsl] = p2_v[k, dsl] + tb_v[0, dsl]
        return _

    lax.fori_loop(0, SPW, bias_body, None)

    def issue_word(c):
        return pltpu.async_copy(
            word_hbm.at[ids_v.at[pl.ds(c * HC, HC)]],
            w_v.at[c % 2], gsem[c % 2])

    gw = {}
    wb = {}
    for c in range(2):
        gw[c] = issue_word(c)

    zeros16 = jnp.zeros((L,), jnp.int32)
    for c in range(NCH):
        j = c % 2
        b, h = divmod(c, NCH // B)
        gw[c].wait()

        def body(i, _, j=j, h=h, c=c):
            tt16 = tt_v[pl.ds(c * HC + (i // L) * L, L)]
            ttb = lax.gather(
                tt16, (zeros16 + (i % L))[:, None],
                dimension_numbers=lax.GatherDimensionNumbers(
                    offset_dims=(), collapsed_slice_dims=(0,),
                    start_index_map=(0,)),
                slice_sizes=(1,),
                mode=lax.GatherScatterMode.PROMISE_IN_BOUNDS)
            ttf = ttb.astype(jnp.float32)
            for d in range(DV):
                dsl = pl.ds(d * L, L)
                w_v[j, i, dsl] = (w_v[j, i, dsl] + p2_v[h * HC + i, dsl]
                                  + ttf * dd_v[dsl])
            return _

        lax.fori_loop(0, HC, body, None)

        wb[c] = pltpu.async_copy(
            w_v.at[j], out_hbm.at[pl.ds(b * S + s0 + h * HC, HC), :], osem[j])
        if c + 2 < NCH:
            wb[c].wait()
            gw[c + 2] = issue_word(c + 2)

    for c in range(NCH - 2, NCH):
        wb[c].wait()


def kernel(input_ids, token_type_ids, word_emb, pos_emb, type_emb):
    ids = input_ids.reshape(-1).astype(jnp.int32)
    tt = token_type_ids.reshape(-1).astype(jnp.int32)
    out = _emb_kernel(ids, tt, word_emb, pos_emb, type_emb)
    return out.reshape(B, S, D)


# speedup vs baseline: 1.7393x; 1.7393x over previous
import functools

import jax
import jax.numpy as jnp
from jax import lax
from jax.experimental import pallas as pl
from jax.experimental.pallas import tpu as pltpu
from jax.experimental.pallas import tpu_sc as plsc

B = 4
S = 2048
D = 768
L = 16
NC = 2
NS = 16
NW = NC * NS
SPW = S // NW
HC = 32
NCH = B * SPW // HC
DV = D // L

_mesh = plsc.VectorSubcoreMesh(core_axis_name="c", subcore_axis_name="s")


@functools.partial(
    pl.kernel,
    mesh=_mesh,
    out_type=jax.ShapeDtypeStruct((B * S, D), jnp.float32),
    scratch_types=[
        pltpu.VMEM((B * SPW,), jnp.int32),
        pltpu.VMEM((B * SPW,), jnp.int32),
        pltpu.VMEM((2, HC, D), jnp.float32),
        pltpu.VMEM((SPW, D), jnp.float32),
        pltpu.VMEM((2, D), jnp.float32),
        pltpu.VMEM((D,), jnp.float32),
        pltpu.SemaphoreType.DMA,
        pltpu.SemaphoreType.DMA,
        pltpu.SemaphoreType.DMA,
        pltpu.SemaphoreType.DMA,
    ],
)
def _emb_kernel(ids_hbm, tt_hbm, word_hbm, pos_hbm, type_hbm, out_hbm,
                ids_v, tt_v, w_v, p2_v, tb_v, dd_v,
                g0, g1, o0, o1):
    gsem = (g0, g1)
    osem = (o0, o1)
    wid = lax.axis_index("s") * NC + lax.axis_index("c")
    s0 = wid * SPW

    for b in range(B):
        pltpu.sync_copy(ids_hbm.at[pl.ds(b * S + s0, SPW)],
                        ids_v.at[pl.ds(b * SPW, SPW)])
        pltpu.sync_copy(tt_hbm.at[pl.ds(b * S + s0, SPW)],
                        tt_v.at[pl.ds(b * SPW, SPW)])
    pltpu.sync_copy(pos_hbm.at[pl.ds(s0, SPW), :], p2_v)
    pltpu.sync_copy(type_hbm, tb_v)

    for d in range(DV):
        dsl = pl.ds(d * L, L)
        dd_v[dsl] = tb_v[1, dsl] - tb_v[0, dsl]

    @plsc.parallel_loop(0, SPW, unroll=1)
    def bias_body(k):
        for d in range(DV):
            dsl = pl.ds(d * L, L)
            p2_v[k, dsl] = p2_v[k, dsl] + tb_v[0, dsl]

    def issue_word(c):
        return pltpu.async_copy(
            word_hbm.at[ids_v.at[pl.ds(c * HC, HC)]],
            w_v.at[c % 2], gsem[c % 2])

    gw = {}
    wb = {}
    for c in range(2):
        gw[c] = issue_word(c)

    zeros16 = jnp.zeros((L,), jnp.int32)
    for c in range(NCH):
        j = c % 2
        b, h = divmod(c, NCH // B)
        gw[c].wait()

        @plsc.parallel_loop(0, HC, unroll=1)
        def body(i, j=j, h=h, c=c):
            tt16 = tt_v[pl.ds(c * HC + (i // L) * L, L)]
            ttb = lax.gather(
                tt16, (zeros16 + (i % L))[:, None],
                dimension_numbers=lax.GatherDimensionNumbers(
                    offset_dims=(), collapsed_slice_dims=(0,),
                    start_index_map=(0,)),
                slice_sizes=(1,),
                mode=lax.GatherScatterMode.PROMISE_IN_BOUNDS)
            ttf = ttb.astype(jnp.float32)
            for d in range(DV):
                dsl = pl.ds(d * L, L)
                w_v[j, i, dsl] = (w_v[j, i, dsl] + p2_v[h * HC + i, dsl]
                                  + ttf * dd_v[dsl])

        wb[c] = pltpu.async_copy(
            w_v.at[j], out_hbm.at[pl.ds(b * S + s0 + h * HC, HC), :], osem[j])
        if c + 2 < NCH:
            wb[c].wait()
            gw[c + 2] = issue_word(c + 2)

    for c in range(NCH - 2, NCH):
        wb[c].wait()


def kernel(input_ids, token_type_ids, word_emb, pos_emb, type_emb):
    ids = input_ids.reshape(-1).astype(jnp.int32)
    tt = token_type_ids.reshape(-1).astype(jnp.int32)
    out = _emb_kernel(ids, tt, word_emb, pos_emb, type_emb)
    return out.reshape(B, S, D)


# speedup vs baseline: 1.8773x; 1.0793x over previous
import functools

import jax
import jax.numpy as jnp
from jax import lax
from jax.experimental import pallas as pl
from jax.experimental.pallas import tpu as pltpu
from jax.experimental.pallas import tpu_sc as plsc

B = 4
S = 2048
D = 768
L = 16
NC = 2
NS = 16
NW = NC * NS
SPW = S // NW
HC = 32
NCH = B * SPW // HC
DV = D // L

_mesh = plsc.VectorSubcoreMesh(core_axis_name="c", subcore_axis_name="s")


@functools.partial(
    pl.kernel,
    mesh=_mesh,
    out_type=jax.ShapeDtypeStruct((B * S, D), jnp.float32),
    scratch_types=[
        pltpu.VMEM((B * SPW,), jnp.int32),
        pltpu.VMEM((B * SPW,), jnp.int32),
        pltpu.VMEM((3, HC, D), jnp.float32),
        pltpu.VMEM((SPW, D), jnp.float32),
        pltpu.VMEM((2, D), jnp.float32),
        pltpu.VMEM((D,), jnp.float32),
        pltpu.SemaphoreType.DMA,
        pltpu.SemaphoreType.DMA,
        pltpu.SemaphoreType.DMA,
        pltpu.SemaphoreType.DMA,
        pltpu.SemaphoreType.DMA,
        pltpu.SemaphoreType.DMA,
    ],
)
def _emb_kernel(ids_hbm, tt_hbm, word_hbm, pos_hbm, type_hbm, out_hbm,
                ids_v, tt_v, w_v, p2_v, tb_v, dd_v,
                g0, g1, g2, o0, o1, o2):
    gsem = (g0, g1, g2)
    osem = (o0, o1, o2)
    wid = lax.axis_index("s") * NC + lax.axis_index("c")
    s0 = wid * SPW

    for b in range(B):
        pltpu.sync_copy(ids_hbm.at[pl.ds(b * S + s0, SPW)],
                        ids_v.at[pl.ds(b * SPW, SPW)])
        pltpu.sync_copy(tt_hbm.at[pl.ds(b * S + s0, SPW)],
                        tt_v.at[pl.ds(b * SPW, SPW)])
    pltpu.sync_copy(pos_hbm.at[pl.ds(s0, SPW), :], p2_v)
    pltpu.sync_copy(type_hbm, tb_v)

    for d in range(DV):
        dsl = pl.ds(d * L, L)
        dd_v[dsl] = tb_v[1, dsl] - tb_v[0, dsl]

    @plsc.parallel_loop(0, SPW * (DV // 16), unroll=1)
    def bias_body(q):
        k = q // (DV // 16)
        d0 = (q % (DV // 16)) * 16
        for d in range(16):
            dsl = pl.ds((d0 + d) * L, L)
            p2_v[k, dsl] = p2_v[k, dsl] + tb_v[0, dsl]

    def issue_word(c):
        return pltpu.async_copy(
            word_hbm.at[ids_v.at[pl.ds(c * HC, HC)]],
            w_v.at[c % 3], gsem[c % 3])

    gw = {}
    wb = {}
    for c in range(2):
        gw[c] = issue_word(c)

    zeros16 = jnp.zeros((L,), jnp.int32)
    for c in range(NCH):
        j = c % 3
        b, h = divmod(c, NCH // B)
        gw[c].wait()

        @plsc.parallel_loop(0, HC, unroll=1)
        def body(i, j=j, h=h, c=c):
            tt16 = tt_v[pl.ds(c * HC + (i // L) * L, L)]
            ttb = lax.gather(
                tt16, (zeros16 + (i % L))[:, None],
                dimension_numbers=lax.GatherDimensionNumbers(
                    offset_dims=(), collapsed_slice_dims=(0,),
                    start_index_map=(0,)),
                slice_sizes=(1,),
                mode=lax.GatherScatterMode.PROMISE_IN_BOUNDS)
            ttf = ttb.astype(jnp.float32)
            for d in range(DV):
                dsl = pl.ds(d * L, L)
                w_v[j, i, dsl] = (w_v[j, i, dsl] + p2_v[h * HC + i, dsl]
                                  + ttf * dd_v[dsl])

        wb[c] = pltpu.async_copy(
            w_v.at[j], out_hbm.at[pl.ds(b * S + s0 + h * HC, HC), :], osem[j])
        if c + 2 < NCH:
            if c >= 1:
                wb[c - 1].wait()
            gw[c + 2] = issue_word(c + 2)

    for c in range(NCH - 3, NCH):
        wb[c].wait()


def kernel(input_ids, token_type_ids, word_emb, pos_emb, type_emb):
    ids = input_ids.reshape(-1).astype(jnp.int32)
    tt = token_type_ids.reshape(-1).astype(jnp.int32)
    out = _emb_kernel(ids, tt, word_emb, pos_emb, type_emb)
    return out.reshape(B, S, D)


# speedup vs baseline: 1.9896x; 1.0598x over previous
import functools

import jax
import jax.numpy as jnp
from jax import lax
from jax.experimental import pallas as pl
from jax.experimental.pallas import tpu as pltpu
from jax.experimental.pallas import tpu_sc as plsc

B = 4
S = 2048
D = 768
L = 16
NC = 2
NS = 16
NW = NC * NS
SPW = S // NW
HC = 32
NCH = B * SPW // HC
DV = D // L

_mesh = plsc.VectorSubcoreMesh(core_axis_name="c", subcore_axis_name="s")


@functools.partial(
    pl.kernel,
    mesh=_mesh,
    out_type=jax.ShapeDtypeStruct((B * S, D), jnp.float32),
    scratch_types=[
        pltpu.VMEM((B * SPW,), jnp.int32),
        pltpu.VMEM((B * SPW,), jnp.int32),
        pltpu.VMEM((3, HC, D), jnp.float32),
        pltpu.VMEM((SPW, D), jnp.float32),
        pltpu.VMEM((2, D), jnp.float32),
        pltpu.VMEM((D,), jnp.float32),
        pltpu.SemaphoreType.DMA,
        pltpu.SemaphoreType.DMA,
        pltpu.SemaphoreType.DMA,
        pltpu.SemaphoreType.DMA,
        pltpu.SemaphoreType.DMA,
        pltpu.SemaphoreType.DMA,
    ],
)
def _emb_kernel(ids_hbm, tt_hbm, word_hbm, pos_hbm, type_hbm, out_hbm,
                ids_v, tt_v, w_v, p2_v, tb_v, dd_v,
                g0, g1, g2, o0, o1, o2):
    gsem = (g0, g1, g2)
    osem = (o0, o1, o2)
    wid = lax.axis_index("s") * NC + lax.axis_index("c")
    s0 = wid * SPW

    cid = pltpu.async_copy(ids_hbm.at[pl.ds(s0, SPW)],
                           ids_v.at[pl.ds(0, SPW)], o0)
    ctt = pltpu.async_copy(tt_hbm.at[pl.ds(s0, SPW)],
                           tt_v.at[pl.ds(0, SPW)], o0)
    cp = pltpu.async_copy(pos_hbm.at[pl.ds(s0, SPW), :], p2_v, o1)
    ctb = pltpu.async_copy(type_hbm, tb_v, o1)

    def issue_word(c):
        return pltpu.async_copy(
            word_hbm.at[ids_v.at[pl.ds(c * HC, HC)]],
            w_v.at[c % 3], gsem[c % 3])

    gw = {}
    wb = {}
    cid.wait()
    ctt.wait()
    for c in range(2):
        gw[c] = issue_word(c)
    for b in range(1, B):
        pltpu.sync_copy(ids_hbm.at[pl.ds(b * S + s0, SPW)],
                        ids_v.at[pl.ds(b * SPW, SPW)])
        pltpu.sync_copy(tt_hbm.at[pl.ds(b * S + s0, SPW)],
                        tt_v.at[pl.ds(b * SPW, SPW)])
    cp.wait()
    ctb.wait()

    for d in range(DV):
        dsl = pl.ds(d * L, L)
        dd_v[dsl] = tb_v[1, dsl] - tb_v[0, dsl]

    @plsc.parallel_loop(0, SPW * (DV // 16), unroll=1)
    def bias_body(q):
        k = q // (DV // 16)
        d0 = (q % (DV // 16)) * 16
        for d in range(16):
            dsl = pl.ds((d0 + d) * L, L)
            p2_v[k, dsl] = p2_v[k, dsl] + tb_v[0, dsl]

    zeros16 = jnp.zeros((L,), jnp.int32)
    for c in range(NCH):
        j = c % 3
        b, h = divmod(c, NCH // B)
        gw[c].wait()

        @plsc.parallel_loop(0, HC, unroll=1)
        def body(i, j=j, h=h, c=c):
            tt16 = tt_v[pl.ds(c * HC + (i // L) * L, L)]
            ttb = lax.gather(
                tt16, (zeros16 + (i % L))[:, None],
                dimension_numbers=lax.GatherDimensionNumbers(
                    offset_dims=(), collapsed_slice_dims=(0,),
                    start_index_map=(0,)),
                slice_sizes=(1,),
                mode=lax.GatherScatterMode.PROMISE_IN_BOUNDS)
            ttf = ttb.astype(jnp.float32)
            for d in range(DV):
                dsl = pl.ds(d * L, L)
                w_v[j, i, dsl] = (w_v[j, i, dsl] + p2_v[h * HC + i, dsl]
                                  + ttf * dd_v[dsl])

        wb[c] = pltpu.async_copy(
            w_v.at[j], out_hbm.at[pl.ds(b * S + s0 + h * HC, HC), :], osem[j])
        if c + 2 < NCH:
            if c >= 1:
                wb[c - 1].wait()
            gw[c + 2] = issue_word(c + 2)

    for c in range(NCH - 3, NCH):
        wb[c].wait()


def kernel(input_ids, token_type_ids, word_emb, pos_emb, type_emb):
    ids = input_ids.reshape(-1).astype(jnp.int32)
    tt = token_type_ids.reshape(-1).astype(jnp.int32)
    out = _emb_kernel(ids, tt, word_emb, pos_emb, type_emb)
    return out.reshape(B, S, D)


# speedup vs baseline: 2.0692x; 1.0400x over previous
import functools

import jax
import jax.numpy as jnp
from jax import lax
from jax.experimental import pallas as pl
from jax.experimental.pallas import tpu as pltpu
from jax.experimental.pallas import tpu_sc as plsc

B = 4
S = 2048
D = 768
L = 16
NC = 2
NS = 16
NW = NC * NS
SPW = S // NW
HC = 32
NCH = B * SPW // HC
DV = D // L

_mesh = plsc.VectorSubcoreMesh(core_axis_name="c", subcore_axis_name="s")


@functools.partial(
    pl.kernel,
    mesh=_mesh,
    out_type=jax.ShapeDtypeStruct((B * S, D), jnp.float32),
    scratch_types=[
        pltpu.VMEM((B * SPW,), jnp.int32),
        pltpu.VMEM((B * SPW,), jnp.int32),
        pltpu.VMEM((3, HC, D), jnp.float32),
        pltpu.VMEM((SPW, D), jnp.float32),
        pltpu.VMEM((2, D), jnp.float32),
        pltpu.VMEM((D,), jnp.float32),
        pltpu.SemaphoreType.DMA,
        pltpu.SemaphoreType.DMA,
        pltpu.SemaphoreType.DMA,
        pltpu.SemaphoreType.DMA,
        pltpu.SemaphoreType.DMA,
        pltpu.SemaphoreType.DMA,
    ],
)
def _emb_kernel(ids_hbm, tt_hbm, word_hbm, pos_hbm, type_hbm, out_hbm,
                ids_v, tt_v, w_v, p2_v, tb_v, dd_v,
                g0, g1, g2, o0, o1, o2):
    gsem = (g0, g1, g2)
    osem = (o0, o1, o2)
    wid = lax.axis_index("s") * NC + lax.axis_index("c")
    s0 = wid * SPW

    cid = pltpu.async_copy(ids_hbm.at[pl.ds(s0, SPW)],
                           ids_v.at[pl.ds(0, SPW)], o0)
    ctt = pltpu.async_copy(tt_hbm.at[pl.ds(s0, SPW)],
                           tt_v.at[pl.ds(0, SPW)], o0)
    cp = pltpu.async_copy(pos_hbm.at[pl.ds(s0, SPW), :], p2_v, o1)
    ctb = pltpu.async_copy(type_hbm, tb_v, o1)

    def issue_word(c):
        return pltpu.async_copy(
            word_hbm.at[ids_v.at[pl.ds(c * HC, HC)]],
            w_v.at[c % 3], gsem[c % 3])

    gw = {}
    wb = {}
    cid.wait()
    ctt.wait()
    for c in range(2):
        gw[c] = issue_word(c)
    for b in range(1, B):
        pltpu.sync_copy(ids_hbm.at[pl.ds(b * S + s0, SPW)],
                        ids_v.at[pl.ds(b * SPW, SPW)])
        pltpu.sync_copy(tt_hbm.at[pl.ds(b * S + s0, SPW)],
                        tt_v.at[pl.ds(b * SPW, SPW)])
    cp.wait()
    ctb.wait()

    dd_vals = [tb_v[1, pl.ds(d * L, L)] - tb_v[0, pl.ds(d * L, L)]
               for d in range(DV)]

    @plsc.parallel_loop(0, SPW * (DV // 16), unroll=1)
    def bias_body(q):
        k = q // (DV // 16)
        d0 = (q % (DV // 16)) * 16
        for d in range(16):
            dsl = pl.ds((d0 + d) * L, L)
            p2_v[k, dsl] = p2_v[k, dsl] + tb_v[0, dsl]

    zeros16 = jnp.zeros((L,), jnp.int32)
    for c in range(NCH):
        j = c % 3
        b, h = divmod(c, NCH // B)
        gw[c].wait()

        @plsc.parallel_loop(0, HC, unroll=1)
        def body(i, j=j, h=h, c=c):
            tt16 = tt_v[pl.ds(c * HC + (i // L) * L, L)]
            ttb = lax.gather(
                tt16, (zeros16 + (i % L))[:, None],
                dimension_numbers=lax.GatherDimensionNumbers(
                    offset_dims=(), collapsed_slice_dims=(0,),
                    start_index_map=(0,)),
                slice_sizes=(1,),
                mode=lax.GatherScatterMode.PROMISE_IN_BOUNDS)
            ttf = ttb.astype(jnp.float32)
            for d in range(DV):
                dsl = pl.ds(d * L, L)
                w_v[j, i, dsl] = (w_v[j, i, dsl] + p2_v[h * HC + i, dsl]
                                  + ttf * dd_vals[d])

        wb[c] = pltpu.async_copy(
            w_v.at[j], out_hbm.at[pl.ds(b * S + s0 + h * HC, HC), :], osem[j])
        if c + 2 < NCH:
            if c >= 1:
                wb[c - 1].wait()
            gw[c + 2] = issue_word(c + 2)

    for c in range(NCH - 3, NCH):
        wb[c].wait()


def kernel(input_ids, token_type_ids, word_emb, pos_emb, type_emb):
    ids = input_ids.reshape(-1).astype(jnp.int32)
    tt = token_type_ids.reshape(-1).astype(jnp.int32)
    out = _emb_kernel(ids, tt, word_emb, pos_emb, type_emb)
    return out.reshape(B, S, D)
